# baseline (speedup 1.0000x reference)
import functools
import jax
import jax.numpy as jnp
from jax import lax
from jax.experimental import pallas as pl
from jax.experimental.pallas import tpu as pltpu
from jax.experimental.pallas import tpu_sc as plsc

D = 128
N = 10000
E = 320000
G = 64
VOCAB = 10
L = 5

TW = 144
EBLK = 2560
NBLK = 2000
EPS = 1e-5

NC = 2
NS = 16
NW = NC * NS
EPW = E // NW
GC = 80
GCH = EPW // GC
SC2 = 80
SCH = EPW // SC2
NPT0 = 624
NPTL = N - 15 * NPT0
P2R = 632
P2T = 40
P2L = P2R - 15 * P2T


def _ln2d(x, g, b):
    m = jnp.mean(x, axis=-1, keepdims=True)
    v = jnp.mean((x - m) ** 2, axis=-1, keepdims=True)
    return (x - m) * jax.lax.rsqrt(v + EPS) * g + b


def _gather_chunk_compute(bufa, bufb, auxb, dstv, srcv, posv, t, lanes):
    def row(i, cy):
        for k in range(8):
            sl = pl.ds(k * 16, 16)
            bufa[i, sl] = bufa[i, sl] + bufb[i, sl]
        return cy

    lax.fori_loop(0, GC, row, 0)

    def grp(j, cy):
        d16 = dstv[pl.ds(t * GC + j * 16, 16)] * 4
        s16 = srcv[pl.ds(t * GC + j * 16, 16)] * 4
        e16 = (lanes + j * 16) * 4
        d2 = jnp.zeros((16,), jnp.float32)
        for comp in range(3):
            pdv = plsc.load_gather(posv, [d16 + comp])
            psv = plsc.load_gather(posv, [s16 + comp])
            df = pdv - psv
            plsc.store_scatter(auxb, [e16 + comp], df)
            d2 = d2 + df * df
        plsc.store_scatter(auxb, [e16 + 3], d2)
        return cy

    lax.fori_loop(0, GC // 16, grp, 0)


def _gather_body(ta, tb, posf, dsth, srch, xout, auxout,
                 dstv, srcv, posv, bufa0, bufb0, bufa1, bufb1, auxb0, auxb1,
                 ga0, gb0, ga1, gb1, wx0, wa0, wx1, wa1):
    c = lax.axis_index("c")
    s = lax.axis_index("s")
    wid = c * NS + s
    ebase = wid * EPW
    pltpu.sync_copy(posf, posv)
    pltpu.sync_copy(dsth.at[pl.ds(ebase, EPW)], dstv)
    pltpu.sync_copy(srch.at[pl.ds(ebase, EPW)], srcv)
    lanes = lax.iota(jnp.int32, 16)

    def g_copies(t, ba, bb, sa, sb):
        off = pl.ds(t * GC, GC)
        return (pltpu.make_async_copy(ta.at[dstv.at[off]], ba, sa),
                pltpu.make_async_copy(tb.at[srcv.at[off]], bb, sb))

    def w_copies(t, ba, ab, sx, sxa):
        base = ebase + t * GC
        return (pltpu.make_async_copy(ba, xout.at[pl.ds(base, GC)], sx),
                pltpu.make_async_copy(ab, auxout.at[pl.ds(base * 4, GC * 4)], sxa))

    a0, b0 = g_copies(0, bufa0, bufb0, ga0, gb0)
    a0.start()
    b0.start()

    def pair(i, carry):
        t0 = 2 * i
        t1 = 2 * i + 1
        t2 = jnp.minimum(t0 + 2, GCH - 1)
        a0, b0 = g_copies(t0, bufa0, bufb0, ga0, gb0)
        a0.wait()
        b0.wait()

        @pl.when(i > 0)
        def _():
            x1, xa1 = w_copies(t0 - 1, bufa1, auxb1, wx1, wa1)
            x1.wait()
            xa1.wait()

        a1, b1 = g_copies(t1, bufa1, bufb1, ga1, gb1)
        a1.start()
        b1.start()
        _gather_chunk_compute(bufa0, bufb0, auxb0, dstv, srcv, posv, t0, lanes)
        x0, xa0 = w_copies(t0, bufa0, auxb0, wx0, wa0)
        x0.start()
        xa0.start()
        a1, b1 = g_copies(t1, bufa1, bufb1, ga1, gb1)
        a1.wait()
        b1.wait()
        x0, xa0 = w_copies(t0, bufa0, auxb0, wx0, wa0)
        x0.wait()
        xa0.wait()
        a2, b2 = g_copies(t2, bufa0, bufb0, ga0, gb0)
        a2.start()
        b2.start()
        _gather_chunk_compute(bufa1, bufb1, auxb1, dstv, srcv, posv, t1, lanes)
        x1, xa1 = w_copies(t1, bufa1, auxb1, wx1, wa1)
        x1.start()
        xa1.start()
        return carry

    lax.fori_loop(0, GCH // 2, pair, 0)
    af, bf = g_copies(GCH - 1, bufa0, bufb0, ga0, gb0)
    af.wait()
    bf.wait()
    xf, xaf = w_copies(GCH - 2, bufa1, auxb1, wx1, wa1)
    xf.wait()
    xaf.wait()
    _gather_chunk_compute(bufa0, bufb0, auxb0, dstv, srcv, posv, GCH - 1, lanes)
    base = ebase + (GCH - 1) * GC
    pltpu.sync_copy(bufa0, xout.at[pl.ds(base, GC)])
    pltpu.sync_copy(auxb0, auxout.at[pl.ds(base * 4, GC * 4)])


def _sc_gather(ta, tb, posflat, dst, src):
    mesh = plsc.VectorSubcoreMesh(core_axis_name="c", subcore_axis_name="s")
    f = pl.kernel(
        _gather_body,
        mesh=mesh,
        out_type=[
            jax.ShapeDtypeStruct((E, D), jnp.float32),
            jax.ShapeDtypeStruct((E * 4,), jnp.float32),
        ],
        scratch_types=[
            pltpu.VMEM((EPW,), jnp.int32),
            pltpu.VMEM((EPW,), jnp.int32),
            pltpu.VMEM((N * 4,), jnp.float32),
            pltpu.VMEM((GC, D), jnp.float32),
            pltpu.VMEM((GC, D), jnp.float32),
            pltpu.VMEM((GC, D), jnp.float32),
            pltpu.VMEM((GC, D), jnp.float32),
            pltpu.VMEM((GC * 4,), jnp.float32),
            pltpu.VMEM((GC * 4,), jnp.float32),
            pltpu.SemaphoreType.DMA,
            pltpu.SemaphoreType.DMA,
            pltpu.SemaphoreType.DMA,
            pltpu.SemaphoreType.DMA,
            pltpu.SemaphoreType.DMA,
            pltpu.SemaphoreType.DMA,
            pltpu.SemaphoreType.DMA,
            pltpu.SemaphoreType.DMA,
        ],
        compiler_params=pltpu.CompilerParams(needs_layout_passes=False),
    )
    return f(ta, tb, posflat, dst, src)


def _scat_pd_build(dsti, pd16b, pd128b, lanes, zero):
    def grp(j, cy):
        d16 = dsti[pl.ds(j * 16, 16)]
        slot = (d16 & 15) * 8
        e16 = lanes + j * 16
        for comp in range(4):
            if zero:
                v = jnp.zeros((16,), jnp.float32)
            else:
                v = plsc.load_gather(pd16b, [e16, jnp.full((16,), comp, jnp.int32)])
            plsc.store_scatter(pd128b, [e16, slot + comp], v)
        return cy

    lax.fori_loop(0, SC2 // 16, grp, 0)


def _scat_shr(dsti, shr):
    def grp(j, cy):
        sl = pl.ds(j * 16, 16)
        shr[sl] = lax.shift_right_logical(dsti[sl], 4)
        return cy

    lax.fori_loop(0, SC2 // 16, grp, 0)


def _scat_chunk(msgh, pdh, dsth, acc, acc2, dsti, shr, msgbuf, pd16b, pd128b,
                lanes, wid, t, nt):
    base = wid * EPW + t * SC2
    pltpu.sync_copy(pdh.at[pl.ds(base, SC2)], pd16b)
    _scat_shr(dsti, shr)
    _scat_pd_build(dsti, pd16b, pd128b, lanes, zero=False)
    pltpu.sync_copy(msgbuf, acc.at[dsti], add=True)
    pltpu.sync_copy(pd128b, acc2.at[shr], add=True)
    _scat_pd_build(dsti, pd16b, pd128b, lanes, zero=True)
    pltpu.sync_copy(dsth.at[pl.ds(wid * EPW + nt * SC2, SC2)], dsti)


def _mscat_body(msgh, pdh, dsth, zerh, mout, pout,
                dsti, shr, msgbuf0, msgbuf1, pd16b, pd128b, acc, acc2,
                sm0, sm1):
    c = lax.axis_index("c")
    s = lax.axis_index("s")
    wid = c * NS + s
    lanes = lax.iota(jnp.int32, 16)

    def zrow(i, cy):
        for k in range(8):
            pd128b[i, pl.ds(k * 16, 16)] = jnp.zeros((16,), jnp.float32)
        return cy

    lax.fori_loop(0, SC2, zrow, 0)

    @pl.when(s < 15)
    def _():
        pltpu.sync_copy(zerh.at[pl.ds(0, NPT0)], acc.at[pl.ds(s * NPT0, NPT0)])
        pltpu.sync_copy(zerh.at[pl.ds(0, P2T)], acc2.at[pl.ds(s * P2T, P2T)])

    @pl.when(s == 15)
    def _():
        pltpu.sync_copy(zerh, acc.at[pl.ds(15 * NPT0, NPTL)])
        pltpu.sync_copy(zerh.at[pl.ds(0, P2L)], acc2.at[pl.ds(15 * P2T, P2L)])

    plsc.subcore_barrier()

    pltpu.sync_copy(dsth.at[pl.ds(wid * EPW, SC2)], dsti)
    pltpu.make_async_copy(msgh.at[pl.ds(wid * EPW, SC2)], msgbuf0, sm0).start()

    def pair(i, carry):
        t0 = 2 * i
        t1 = 2 * i + 1
        t2 = jnp.minimum(t0 + 2, SCH - 1)
        b0 = wid * EPW + t0 * SC2
        b1 = wid * EPW + t1 * SC2
        b2 = wid * EPW + t2 * SC2
        pltpu.make_async_copy(msgh.at[pl.ds(b0, SC2)], msgbuf0, sm0).wait()
        pltpu.make_async_copy(msgh.at[pl.ds(b1, SC2)], msgbuf1, sm1).start()
        _scat_chunk(msgh, pdh, dsth, acc, acc2, dsti, shr, msgbuf0, pd16b,
                    pd128b, lanes, wid, t0, t1)
        pltpu.make_async_copy(msgh.at[pl.ds(b1, SC2)], msgbuf1, sm1).wait()
        pltpu.make_async_copy(msgh.at[pl.ds(b2, SC2)], msgbuf0, sm0).start()
        _scat_chunk(msgh, pdh, dsth, acc, acc2, dsti, shr, msgbuf1, pd16b,
                    pd128b, lanes, wid, t1, t2)
        return carry

    lax.fori_loop(0, SCH // 2, pair, 0)
    pltpu.make_async_copy(msgh.at[pl.ds(wid * EPW + (SCH - 1) * SC2, SC2)],
                          msgbuf0, sm0).wait()
    _scat_chunk(msgh, pdh, dsth, acc, acc2, dsti, shr, msgbuf0, pd16b,
                pd128b, lanes, wid, SCH - 1, SCH - 1)
    plsc.subcore_barrier()

    @pl.when(s < 15)
    def _():
        pltpu.sync_copy(acc.at[pl.ds(s * NPT0, NPT0)],
                        mout.at[pl.ds(c * N + s * NPT0, NPT0)])
        pltpu.sync_copy(acc2.at[pl.ds(s * P2T, P2T)],
                        pout.at[pl.ds(c * P2R + s * P2T, P2T)])

    @pl.when(s == 15)
    def _():
        pltpu.sync_copy(acc.at[pl.ds(15 * NPT0, NPTL)],
                        mout.at[pl.ds(c * N + 15 * NPT0, NPTL)])
        pltpu.sync_copy(acc2.at[pl.ds(15 * P2T, P2L)],
                        pout.at[pl.ds(c * P2R + 15 * P2T, P2L)])


def _sc_mscat(msg, pd16, dst, zer):
    mesh = plsc.VectorSubcoreMesh(core_axis_name="c", subcore_axis_name="s")
    f = pl.kernel(
        _mscat_body,
        mesh=mesh,
        out_type=[
            jax.ShapeDtypeStruct((NC * N, D), jnp.float32),
            jax.ShapeDtypeStruct((NC * P2R, D), jnp.float32),
        ],
        scratch_types=[
            pltpu.VMEM((SC2,), jnp.int32),
            pltpu.VMEM((SC2,), jnp.int32),
            pltpu.VMEM((SC2, D), jnp.float32),
            pltpu.VMEM((SC2, D), jnp.float32),
            pltpu.VMEM((SC2, 8), jnp.float32),
            pltpu.VMEM((SC2, D), jnp.float32),
            pltpu.VMEM_SHARED((N, D), jnp.float32),
            pltpu.VMEM_SHARED((P2R, D), jnp.float32),
            pltpu.SemaphoreType.DMA,
            pltpu.SemaphoreType.DMA,
        ],
        compiler_params=pltpu.CompilerParams(needs_layout_passes=False),
    )
    return f(msg, pd16, dst, zer)


def _init_body(atoms_ref, emb_ref, wd_ref, ws_ref, h_ref, ta_ref, tb_ref):
    at = atoms_ref[...]
    oh = (at == jax.lax.broadcasted_iota(jnp.int32, (1, VOCAB), 1)).astype(jnp.float32)
    h = jnp.dot(oh, emb_ref[...], preferred_element_type=jnp.float32)
    h_ref[...] = h
    ta_ref[...] = jnp.dot(h, wd_ref[...], preferred_element_type=jnp.float32)
    tb_ref[...] = jnp.dot(h, ws_ref[...], preferred_element_type=jnp.float32)


def _init_call(atoms2d, emb, wd, ws):
    grid = N // NBLK
    cst = lambda i: (0, 0)
    return pl.pallas_call(
        _init_body,
        grid=(grid,),
        in_specs=[
            pl.BlockSpec((NBLK, 1), lambda i: (i, 0)),
            pl.BlockSpec((VOCAB, D), cst),
            pl.BlockSpec((D, D), cst),
            pl.BlockSpec((D, D), cst),
        ],
        out_specs=[
            pl.BlockSpec((NBLK, D), lambda i: (i, 0)),
            pl.BlockSpec((NBLK, D), lambda i: (i, 0)),
            pl.BlockSpec((NBLK, D), lambda i: (i, 0)),
        ],
        out_shape=[
            jax.ShapeDtypeStruct((N, D), jnp.float32),
            jax.ShapeDtypeStruct((N, D), jnp.float32),
            jax.ShapeDtypeStruct((N, D), jnp.float32),
        ],
        compiler_params=pltpu.CompilerParams(
            dimension_semantics=("parallel",)),
    )(atoms2d, emb, wd, ws)


def _edge_body(xp_ref, aux_ref, wdist_ref, b1_ref, g1_ref, bn1_ref,
               w2_ref, b2_ref, g2_ref, bn2_ref,
               p1_ref, pb1_ref, pg_ref, pbn_ref, p2_ref, pb2_ref,
               msg_ref, pd_ref):
    xp = xp_ref[...]
    aux = aux_ref[...]
    diff = aux[:, :3]
    dist = jnp.sqrt(aux[:, 3:4])
    x = xp + dist * wdist_ref[...] + b1_ref[...]
    x = jax.nn.relu(_ln2d(x, g1_ref[...], bn1_ref[...]))
    x = jnp.dot(x, w2_ref[...], preferred_element_type=jnp.float32) + b2_ref[...]
    msg = jax.nn.relu(_ln2d(x, g2_ref[...], bn2_ref[...]))
    msg_ref[...] = msg
    p = jnp.dot(msg, p1_ref[...], preferred_element_type=jnp.float32) + pb1_ref[...]
    p = jax.nn.relu(_ln2d(p, pg_ref[...], pbn_ref[...]))
    pw = jnp.dot(p, p2_ref[...], preferred_element_type=jnp.float32) + pb2_ref[...]
    ones = jnp.ones((xp.shape[0], 1), jnp.float32)
    z4 = jnp.zeros((xp.shape[0], 4), jnp.float32)
    pd_ref[...] = jnp.concatenate([diff * pw, ones, z4], axis=-1)


def _edge_call(xpre, aux, lp):
    grid = E // EBLK
    row = lambda i: (i, 0)
    cst = lambda i: (0, 0)
    wdist = lp["msg_l1"]["w"][2 * D:2 * D + 1, :]
    args = [
        xpre, aux,
        wdist, lp["msg_l1"]["b"][None, :], lp["msg_n1"]["g"][None, :], lp["msg_n1"]["b"][None, :],
        lp["msg_l2"]["w"], lp["msg_l2"]["b"][None, :], lp["msg_n2"]["g"][None, :], lp["msg_n2"]["b"][None, :],
        lp["pos_l1"]["w"], lp["pos_l1"]["b"][None, :], lp["pos_n1"]["g"][None, :], lp["pos_n1"]["b"][None, :],
        lp["pos_l2"]["w"], lp["pos_l2"]["b"][None, :],
    ]
    in_specs = [
        pl.BlockSpec((EBLK, D), row),
        pl.BlockSpec((EBLK, 4), row),
        pl.BlockSpec((1, D), cst), pl.BlockSpec((1, D), cst),
        pl.BlockSpec((1, D), cst), pl.BlockSpec((1, D), cst),
        pl.BlockSpec((D, D), cst), pl.BlockSpec((1, D), cst),
        pl.BlockSpec((1, D), cst), pl.BlockSpec((1, D), cst),
        pl.BlockSpec((D, D), cst), pl.BlockSpec((1, D), cst),
        pl.BlockSpec((1, D), cst), pl.BlockSpec((1, D), cst),
        pl.BlockSpec((D, 1), cst), pl.BlockSpec((1, 1), cst),
    ]
    return pl.pallas_call(
        _edge_body,
        grid=(grid,),
        in_specs=in_specs,
        out_specs=[
            pl.BlockSpec((EBLK, D), row),
            pl.BlockSpec((EBLK, 8), row),
        ],
        out_shape=[
            jax.ShapeDtypeStruct((E, D), jnp.float32),
            jax.ShapeDtypeStruct((E, 8), jnp.float32),
        ],
        compiler_params=pltpu.CompilerParams(
            dimension_semantics=("parallel",)),
    )(*args)


def _node_body(h_ref, macc_ref, pacc_ref, pos_ref,
               u1h_ref, u1m_ref, ub1_ref, ug1_ref, ubn1_ref,
               u2_ref, ub2_ref, ug2_ref, ubn2_ref,
               wd_ref, ws_ref, h_out, pos_out, *outs, last):
    h = h_ref[...]
    macc = macc_ref[...][0] + macc_ref[...][1]
    pacc = pacc_ref[...][0] + pacc_ref[...][1]
    cnt = jnp.maximum(pacc[:, 3:4], 1.0)
    pos_aggr = pacc[:, :3] / cnt
    x = (jnp.dot(h, u1h_ref[...], preferred_element_type=jnp.float32)
         + jnp.dot(macc, u1m_ref[...], preferred_element_type=jnp.float32)
         + ub1_ref[...])
    x = jax.nn.relu(_ln2d(x, ug1_ref[...], ubn1_ref[...]))
    x = jnp.dot(x, u2_ref[...], preferred_element_type=jnp.float32) + ub2_ref[...]
    x = jax.nn.relu(_ln2d(x, ug2_ref[...], ubn2_ref[...]))
    hn = h + x
    h_out[...] = hn
    pos = pos_ref[...]
    zero = jnp.zeros((pos.shape[0], 1), jnp.float32)
    posn = pos + jnp.concatenate([pos_aggr, zero], axis=-1)
    pos_out[...] = posn
    if not last:
        ta_out, tb_out = outs
        ta_out[...] = jnp.dot(hn, wd_ref[...], preferred_element_type=jnp.float32)
        tb_out[...] = jnp.dot(hn, ws_ref[...], preferred_element_type=jnp.float32)


def _node_call(h, macc2, pacc, pos4, lp, nxt_wd, nxt_ws, last):
    grid = N // NBLK
    row = lambda i: (i, 0)
    cst = lambda i: (0, 0)
    args = [
        h, macc2, pacc, pos4,
        lp["upd_l1"]["w"][:D, :], lp["upd_l1"]["w"][D:, :], lp["upd_l1"]["b"][None, :],
        lp["upd_n1"]["g"][None, :], lp["upd_n1"]["b"][None, :],
        lp["upd_l2"]["w"], lp["upd_l2"]["b"][None, :],
        lp["upd_n2"]["g"][None, :], lp["upd_n2"]["b"][None, :],
        nxt_wd, nxt_ws,
    ]
    in_specs = [
        pl.BlockSpec((NBLK, D), row),
        pl.BlockSpec((NC, NBLK, D), lambda i: (0, i, 0)),
        pl.BlockSpec((NC, NBLK, 8), lambda i: (0, i, 0)),
        pl.BlockSpec((NBLK, 4), row),
        pl.BlockSpec((D, D), cst), pl.BlockSpec((D, D), cst), pl.BlockSpec((1, D), cst),
        pl.BlockSpec((1, D), cst), pl.BlockSpec((1, D), cst),
        pl.BlockSpec((D, D), cst), pl.BlockSpec((1, D), cst),
        pl.BlockSpec((1, D), cst), pl.BlockSpec((1, D), cst),
        pl.BlockSpec((D, D), cst), pl.BlockSpec((D, D), cst),
    ]
    out_specs = [pl.BlockSpec((NBLK, D), row), pl.BlockSpec((NBLK, 4), row)]
    out_shape = [jax.ShapeDtypeStruct((N, D), jnp.float32),
                 jax.ShapeDtypeStruct((N, 4), jnp.float32)]
    if not last:
        out_specs += [pl.BlockSpec((NBLK, D), row), pl.BlockSpec((NBLK, D), row)]
        out_shape += [jax.ShapeDtypeStruct((N, D), jnp.float32),
                      jax.ShapeDtypeStruct((N, D), jnp.float32)]
    res = pl.pallas_call(
        functools.partial(_node_body, last=last),
        grid=(grid,),
        in_specs=in_specs,
        out_specs=out_specs,
        out_shape=out_shape,
        compiler_params=pltpu.CompilerParams(
            dimension_semantics=("parallel",)),
    )(*args)
    if last:
        return res[0], res[1], None, None
    return res


def _pool_body(h_ref, bt_ref, p1_ref, pb1_ref, p2_ref, pb2_ref,
               out_ref, acc_ref):
    i = pl.program_id(0)

    @pl.when(i == 0)
    def _():
        acc_ref[...] = jnp.zeros_like(acc_ref)

    bt = bt_ref[...].reshape(1, -1)
    gi = jax.lax.broadcasted_iota(jnp.int32, (G, 1), 0)
    mask = (bt == gi).astype(jnp.float32)
    acc_ref[...] += jnp.dot(mask, h_ref[...], preferred_element_type=jnp.float32)

    @pl.when(i == pl.num_programs(0) - 1)
    def _():
        pooled = acc_ref[...]
        x = jax.nn.relu(jnp.dot(pooled, p1_ref[...],
                                preferred_element_type=jnp.float32) + pb1_ref[...])
        out_ref[...] = jnp.dot(x, p2_ref[...],
                               preferred_element_type=jnp.float32) + pb2_ref[...]


def _pool_call(h, batch3d, params):
    grid = N // NBLK
    cst = lambda i: (0, 0)
    return pl.pallas_call(
        _pool_body,
        grid=(grid,),
        in_specs=[
            pl.BlockSpec((NBLK, D), lambda i: (i, 0)),
            pl.BlockSpec((1, 1, NBLK), lambda i: (i, 0, 0)),
            pl.BlockSpec((D, D), cst), pl.BlockSpec((1, D), cst),
            pl.BlockSpec((D, 1), cst), pl.BlockSpec((1, 1), cst),
        ],
        out_specs=pl.BlockSpec((G, 1), cst),
        out_shape=jax.ShapeDtypeStruct((G, 1), jnp.float32),
        scratch_shapes=[pltpu.VMEM((G, D), jnp.float32)],
        compiler_params=pltpu.CompilerParams(
            dimension_semantics=("arbitrary",)),
    )(h, batch3d, params["pred_l1"]["w"], params["pred_l1"]["b"][None, :],
      params["pred_l2"]["w"], params["pred_l2"]["b"][None, :])


def kernel(atoms, pos, edge_index, batch, params):
    src = edge_index[0]
    dst = edge_index[1]
    atoms2d = atoms[:, None]
    batch3d = batch.reshape(N // NBLK, 1, NBLK)
    pos4 = jnp.pad(pos, ((0, 0), (0, 1)))
    zer = jnp.zeros((NPTL, D), jnp.float32)

    layers = params["layers"]
    h, ta, tb = _init_call(atoms2d, params["emb"],
                           layers[0]["msg_l1"]["w"][:D, :],
                           layers[0]["msg_l1"]["w"][D:2 * D, :])

    for li in range(L):
        lp = layers[li]
        xpre, auxflat = _sc_gather(ta, tb, pos4.reshape(-1), dst, src)
        msg, pd = _edge_call(xpre, auxflat.reshape(E, 4), lp)
        maccf, paccf = _sc_mscat(msg, pd, dst, zer)
        macc2 = maccf.reshape(NC, N, D)
        pacc = paccf.reshape(NC, P2R, D)[:, :N // 16].reshape(NC, N, 8)

        last = li == L - 1
        if last:
            nxt_wd = jnp.zeros((D, D), jnp.float32)
            nxt_ws = jnp.zeros((D, D), jnp.float32)
        else:
            nxt_wd = layers[li + 1]["msg_l1"]["w"][:D, :]
            nxt_ws = layers[li + 1]["msg_l1"]["w"][D:2 * D, :]
        h, pos4, ta, tb = _node_call(h, macc2, pacc, pos4, lp, nxt_wd, nxt_ws, last)

    return _pool_call(h, batch3d, params)


# speedup vs baseline: 1.0687x; 1.0687x over previous
import functools
import jax
import jax.numpy as jnp
from jax import lax
from jax.experimental import pallas as pl
from jax.experimental.pallas import tpu as pltpu
from jax.experimental.pallas import tpu_sc as plsc

D = 128
N = 10000
E = 320000
G = 64
VOCAB = 10
L = 5

TW = 144
EBLK = 2560
NBLK = 2000
EPS = 1e-5

NC = 2
NS = 16
NW = NC * NS
EPW = E // NW
GC = 80
GCH = EPW // GC
SC2 = 80
SCH = EPW // SC2
NPT0 = 624
NPTL = N - 15 * NPT0
N4P = 40960
PDB = 8192
PC = 400


def _ln2d(x, g, b):
    m = jnp.mean(x, axis=-1, keepdims=True)
    v = jnp.mean((x - m) ** 2, axis=-1, keepdims=True)
    return (x - m) * jax.lax.rsqrt(v + EPS) * g + b


def _gather_chunk_compute(bufa, bufb, auxb, dstv, srcv, posv, t, lanes):
    def row(i, cy):
        for k in range(8):
            sl = pl.ds(k * 16, 16)
            bufa[i, sl] = bufa[i, sl] + bufb[i, sl]
        return cy

    lax.fori_loop(0, GC, row, 0)

    def grp(j, cy):
        d16 = dstv[pl.ds(t * GC + j * 16, 16)] * 4
        s16 = srcv[pl.ds(t * GC + j * 16, 16)] * 4
        e16 = (lanes + j * 16) * 4
        d2 = jnp.zeros((16,), jnp.float32)
        for comp in range(3):
            pdv = plsc.load_gather(posv, [d16 + comp])
            psv = plsc.load_gather(posv, [s16 + comp])
            df = pdv - psv
            plsc.store_scatter(auxb, [e16 + comp], df)
            d2 = d2 + df * df
        plsc.store_scatter(auxb, [e16 + 3], d2)
        return cy

    lax.fori_loop(0, GC // 16, grp, 0)


def _gather_body(ta, tb, posf, dsth, srch, xout, auxout,
                 dstv, srcv, posv, bufa0, bufb0, bufa1, bufb1, auxb0, auxb1,
                 ga0, gb0, ga1, gb1, wx0, wa0, wx1, wa1):
    c = lax.axis_index("c")
    s = lax.axis_index("s")
    wid = c * NS + s
    ebase = wid * EPW
    pltpu.sync_copy(posf, posv)
    pltpu.sync_copy(dsth.at[pl.ds(ebase, EPW)], dstv)
    pltpu.sync_copy(srch.at[pl.ds(ebase, EPW)], srcv)
    lanes = lax.iota(jnp.int32, 16)

    def g_copies(t, ba, bb, sa, sb):
        off = pl.ds(t * GC, GC)
        return (pltpu.make_async_copy(ta.at[dstv.at[off]], ba, sa),
                pltpu.make_async_copy(tb.at[srcv.at[off]], bb, sb))

    def w_copies(t, ba, ab, sx, sxa):
        base = ebase + t * GC
        return (pltpu.make_async_copy(ba, xout.at[pl.ds(base, GC)], sx),
                pltpu.make_async_copy(ab, auxout.at[pl.ds(base * 4, GC * 4)], sxa))

    a0, b0 = g_copies(0, bufa0, bufb0, ga0, gb0)
    a0.start()
    b0.start()

    def pair(i, carry):
        t0 = 2 * i
        t1 = 2 * i + 1
        t2 = jnp.minimum(t0 + 2, GCH - 1)
        a0, b0 = g_copies(t0, bufa0, bufb0, ga0, gb0)
        a0.wait()
        b0.wait()

        @pl.when(i > 0)
        def _():
            x1, xa1 = w_copies(t0 - 1, bufa1, auxb1, wx1, wa1)
            x1.wait()
            xa1.wait()

        a1, b1 = g_copies(t1, bufa1, bufb1, ga1, gb1)
        a1.start()
        b1.start()
        _gather_chunk_compute(bufa0, bufb0, auxb0, dstv, srcv, posv, t0, lanes)
        x0, xa0 = w_copies(t0, bufa0, auxb0, wx0, wa0)
        x0.start()
        xa0.start()
        a1, b1 = g_copies(t1, bufa1, bufb1, ga1, gb1)
        a1.wait()
        b1.wait()
        x0, xa0 = w_copies(t0, bufa0, auxb0, wx0, wa0)
        x0.wait()
        xa0.wait()
        a2, b2 = g_copies(t2, bufa0, bufb0, ga0, gb0)
        a2.start()
        b2.start()
        _gather_chunk_compute(bufa1, bufb1, auxb1, dstv, srcv, posv, t1, lanes)
        x1, xa1 = w_copies(t1, bufa1, auxb1, wx1, wa1)
        x1.start()
        xa1.start()
        return carry

    lax.fori_loop(0, GCH // 2, pair, 0)
    af, bf = g_copies(GCH - 1, bufa0, bufb0, ga0, gb0)
    af.wait()
    bf.wait()
    xf, xaf = w_copies(GCH - 2, bufa1, auxb1, wx1, wa1)
    xf.wait()
    xaf.wait()
    _gather_chunk_compute(bufa0, bufb0, auxb0, dstv, srcv, posv, GCH - 1, lanes)
    base = ebase + (GCH - 1) * GC
    pltpu.sync_copy(bufa0, xout.at[pl.ds(base, GC)])
    pltpu.sync_copy(auxb0, auxout.at[pl.ds(base * 4, GC * 4)])


def _sc_gather(ta, tb, posflat, dst, src):
    mesh = plsc.VectorSubcoreMesh(core_axis_name="c", subcore_axis_name="s")
    f = pl.kernel(
        _gather_body,
        mesh=mesh,
        out_type=[
            jax.ShapeDtypeStruct((E, D), jnp.float32),
            jax.ShapeDtypeStruct((E * 4,), jnp.float32),
        ],
        scratch_types=[
            pltpu.VMEM((EPW,), jnp.int32),
            pltpu.VMEM((EPW,), jnp.int32),
            pltpu.VMEM((N * 4,), jnp.float32),
            pltpu.VMEM((GC, D), jnp.float32),
            pltpu.VMEM((GC, D), jnp.float32),
            pltpu.VMEM((GC, D), jnp.float32),
            pltpu.VMEM((GC, D), jnp.float32),
            pltpu.VMEM((GC * 4,), jnp.float32),
            pltpu.VMEM((GC * 4,), jnp.float32),
            pltpu.SemaphoreType.DMA,
            pltpu.SemaphoreType.DMA,
            pltpu.SemaphoreType.DMA,
            pltpu.SemaphoreType.DMA,
            pltpu.SemaphoreType.DMA,
            pltpu.SemaphoreType.DMA,
            pltpu.SemaphoreType.DMA,
            pltpu.SemaphoreType.DMA,
        ],
        compiler_params=pltpu.CompilerParams(needs_layout_passes=False),
    )
    return f(ta, tb, posflat, dst, src)


def _mscat_body(msgh, dsth, zerh, mout, dsti0, dsti1, msgbuf0, msgbuf1, acc,
                sm0, sm1):
    c = lax.axis_index("c")
    s = lax.axis_index("s")
    wid = c * NS + s

    @pl.when(s < 15)
    def _():
        pltpu.sync_copy(zerh.at[pl.ds(0, NPT0)], acc.at[pl.ds(s * NPT0, NPT0)])

    @pl.when(s == 15)
    def _():
        pltpu.sync_copy(zerh, acc.at[pl.ds(15 * NPT0, NPTL)])

    plsc.subcore_barrier()

    pltpu.sync_copy(dsth.at[pl.ds(wid * EPW, SC2)], dsti0)
    pltpu.make_async_copy(msgh.at[pl.ds(wid * EPW, SC2)], msgbuf0, sm0).start()

    def pair(i, carry):
        t0 = 2 * i
        t1 = 2 * i + 1
        t2 = jnp.minimum(t0 + 2, SCH - 1)
        b0 = wid * EPW + t0 * SC2
        b1 = wid * EPW + t1 * SC2
        b2 = wid * EPW + t2 * SC2
        pltpu.make_async_copy(msgh.at[pl.ds(b0, SC2)], msgbuf0, sm0).wait()
        pltpu.sync_copy(dsth.at[pl.ds(b1, SC2)], dsti1)
        pltpu.make_async_copy(msgh.at[pl.ds(b1, SC2)], msgbuf1, sm1).start()
        pltpu.sync_copy(msgbuf0, acc.at[dsti0], add=True)
        pltpu.make_async_copy(msgh.at[pl.ds(b1, SC2)], msgbuf1, sm1).wait()
        pltpu.sync_copy(dsth.at[pl.ds(b2, SC2)], dsti0)
        pltpu.make_async_copy(msgh.at[pl.ds(b2, SC2)], msgbuf0, sm0).start()
        pltpu.sync_copy(msgbuf1, acc.at[dsti1], add=True)
        return carry

    lax.fori_loop(0, SCH // 2, pair, 0)
    pltpu.make_async_copy(msgh.at[pl.ds(wid * EPW + (SCH - 1) * SC2, SC2)],
                          msgbuf0, sm0).wait()
    pltpu.sync_copy(msgbuf0, acc.at[dsti0], add=True)
    plsc.subcore_barrier()

    @pl.when(s < 15)
    def _():
        pltpu.sync_copy(acc.at[pl.ds(s * NPT0, NPT0)],
                        mout.at[pl.ds(c * N + s * NPT0, NPT0)])

    @pl.when(s == 15)
    def _():
        pltpu.sync_copy(acc.at[pl.ds(15 * NPT0, NPTL)],
                        mout.at[pl.ds(c * N + 15 * NPT0, NPTL)])


def _sc_mscat(msg, dst, zer):
    mesh = plsc.VectorSubcoreMesh(core_axis_name="c", subcore_axis_name="s")
    f = pl.kernel(
        _mscat_body,
        mesh=mesh,
        out_type=jax.ShapeDtypeStruct((NC * N, D), jnp.float32),
        scratch_types=[
            pltpu.VMEM((SC2,), jnp.int32),
            pltpu.VMEM((SC2,), jnp.int32),
            pltpu.VMEM((SC2, D), jnp.float32),
            pltpu.VMEM((SC2, D), jnp.float32),
            pltpu.VMEM_SHARED((N, D), jnp.float32),
            pltpu.SemaphoreType.DMA,
            pltpu.SemaphoreType.DMA,
        ],
        compiler_params=pltpu.CompilerParams(needs_layout_passes=False),
    )
    return f(msg, dst, zer)


def _pscat_body(pdh, dsth, pout, dstv, pdbuf, pdacc):
    c = lax.axis_index("c")
    s = lax.axis_index("s")
    wid = c * NS + s
    lanes = lax.iota(jnp.int32, 16)

    def z1(i, cy):
        pdacc[pl.ds(i * 16, 16)] = jnp.zeros((16,), jnp.float32)
        return cy

    lax.fori_loop(0, N4P // 16, z1, 0)
    pltpu.sync_copy(dsth.at[pl.ds(wid * EPW, EPW)], dstv)

    def chunk(t, carry):
        base = wid * EPW + t * PC
        pltpu.sync_copy(pdh.at[pl.ds(base, PC)], pdbuf)

        def grp(j, cy):
            d16 = dstv[pl.ds(t * PC + j * 16, 16)] * 4
            e16 = lanes + j * 16
            for comp in range(4):
                v = plsc.load_gather(pdbuf, [e16, jnp.full((16,), comp, jnp.int32)])
                plsc.addupdate_scatter(pdacc, [d16 + comp], v)
            return cy

        lax.fori_loop(0, PC // 16, grp, 0)
        return carry

    lax.fori_loop(0, EPW // PC, chunk, 0)
    pltpu.sync_copy(pdacc, pout.at[pl.ds(wid * N4P, N4P)])


def _sc_pscat(pd8, dst):
    mesh = plsc.VectorSubcoreMesh(core_axis_name="c", subcore_axis_name="s")
    f = pl.kernel(
        _pscat_body,
        mesh=mesh,
        out_type=jax.ShapeDtypeStruct((NW * N4P,), jnp.float32),
        scratch_types=[
            pltpu.VMEM((EPW,), jnp.int32),
            pltpu.VMEM((PC, 8), jnp.float32),
            pltpu.VMEM((N4P,), jnp.float32),
        ],
        compiler_params=pltpu.CompilerParams(needs_layout_passes=False),
    )
    return f(pd8, dst)


def _pdred_body(p_ref, o_ref):
    o_ref[...] = jnp.sum(p_ref[...], axis=0, keepdims=True)[None]


def _pdred_call(p2d):
    grid = N4P // PDB
    return pl.pallas_call(
        _pdred_body,
        grid=(grid,),
        in_specs=[pl.BlockSpec((NW, PDB), lambda i: (0, i))],
        out_specs=pl.BlockSpec((1, 1, PDB), lambda i: (i, 0, 0)),
        out_shape=jax.ShapeDtypeStruct((grid, 1, PDB), jnp.float32),
        compiler_params=pltpu.CompilerParams(
            dimension_semantics=("parallel",)),
    )(p2d)


def _init_body(atoms_ref, emb_ref, wd_ref, ws_ref, h_ref, ta_ref, tb_ref):
    at = atoms_ref[...]
    oh = (at == jax.lax.broadcasted_iota(jnp.int32, (1, VOCAB), 1)).astype(jnp.float32)
    h = jnp.dot(oh, emb_ref[...], preferred_element_type=jnp.float32)
    h_ref[...] = h
    ta_ref[...] = jnp.dot(h, wd_ref[...], preferred_element_type=jnp.float32)
    tb_ref[...] = jnp.dot(h, ws_ref[...], preferred_element_type=jnp.float32)


def _init_call(atoms2d, emb, wd, ws):
    grid = N // NBLK
    cst = lambda i: (0, 0)
    return pl.pallas_call(
        _init_body,
        grid=(grid,),
        in_specs=[
            pl.BlockSpec((NBLK, 1), lambda i: (i, 0)),
            pl.BlockSpec((VOCAB, D), cst),
            pl.BlockSpec((D, D), cst),
            pl.BlockSpec((D, D), cst),
        ],
        out_specs=[
            pl.BlockSpec((NBLK, D), lambda i: (i, 0)),
            pl.BlockSpec((NBLK, D), lambda i: (i, 0)),
            pl.BlockSpec((NBLK, D), lambda i: (i, 0)),
        ],
        out_shape=[
            jax.ShapeDtypeStruct((N, D), jnp.float32),
            jax.ShapeDtypeStruct((N, D), jnp.float32),
            jax.ShapeDtypeStruct((N, D), jnp.float32),
        ],
        compiler_params=pltpu.CompilerParams(
            dimension_semantics=("parallel",)),
    )(atoms2d, emb, wd, ws)


def _edge_body(xp_ref, aux_ref, wdist_ref, b1_ref, g1_ref, bn1_ref,
               w2_ref, b2_ref, g2_ref, bn2_ref,
               p1_ref, pb1_ref, pg_ref, pbn_ref, p2_ref, pb2_ref,
               msg_ref, pd_ref):
    xp = xp_ref[...]
    aux = aux_ref[...]
    diff = aux[:, :3]
    dist = jnp.sqrt(aux[:, 3:4])
    x = xp + dist * wdist_ref[...] + b1_ref[...]
    x = jax.nn.relu(_ln2d(x, g1_ref[...], bn1_ref[...]))
    x = jnp.dot(x, w2_ref[...], preferred_element_type=jnp.float32) + b2_ref[...]
    msg = jax.nn.relu(_ln2d(x, g2_ref[...], bn2_ref[...]))
    msg_ref[...] = msg
    p = jnp.dot(msg, p1_ref[...], preferred_element_type=jnp.float32) + pb1_ref[...]
    p = jax.nn.relu(_ln2d(p, pg_ref[...], pbn_ref[...]))
    pw = jnp.dot(p, p2_ref[...], preferred_element_type=jnp.float32) + pb2_ref[...]
    ones = jnp.ones((xp.shape[0], 1), jnp.float32)
    z4 = jnp.zeros((xp.shape[0], 4), jnp.float32)
    pd_ref[...] = jnp.concatenate([diff * pw, ones, z4], axis=-1)


def _edge_call(xpre, aux, lp):
    grid = E // EBLK
    row = lambda i: (i, 0)
    cst = lambda i: (0, 0)
    wdist = lp["msg_l1"]["w"][2 * D:2 * D + 1, :]
    args = [
        xpre, aux,
        wdist, lp["msg_l1"]["b"][None, :], lp["msg_n1"]["g"][None, :], lp["msg_n1"]["b"][None, :],
        lp["msg_l2"]["w"], lp["msg_l2"]["b"][None, :], lp["msg_n2"]["g"][None, :], lp["msg_n2"]["b"][None, :],
        lp["pos_l1"]["w"], lp["pos_l1"]["b"][None, :], lp["pos_n1"]["g"][None, :], lp["pos_n1"]["b"][None, :],
        lp["pos_l2"]["w"], lp["pos_l2"]["b"][None, :],
    ]
    in_specs = [
        pl.BlockSpec((EBLK, D), row),
        pl.BlockSpec((EBLK, 4), row),
        pl.BlockSpec((1, D), cst), pl.BlockSpec((1, D), cst),
        pl.BlockSpec((1, D), cst), pl.BlockSpec((1, D), cst),
        pl.BlockSpec((D, D), cst), pl.BlockSpec((1, D), cst),
        pl.BlockSpec((1, D), cst), pl.BlockSpec((1, D), cst),
        pl.BlockSpec((D, D), cst), pl.BlockSpec((1, D), cst),
        pl.BlockSpec((1, D), cst), pl.BlockSpec((1, D), cst),
        pl.BlockSpec((D, 1), cst), pl.BlockSpec((1, 1), cst),
    ]
    return pl.pallas_call(
        _edge_body,
        grid=(grid,),
        in_specs=in_specs,
        out_specs=[
            pl.BlockSpec((EBLK, D), row),
            pl.BlockSpec((EBLK, 8), row),
        ],
        out_shape=[
            jax.ShapeDtypeStruct((E, D), jnp.float32),
            jax.ShapeDtypeStruct((E, 8), jnp.float32),
        ],
        compiler_params=pltpu.CompilerParams(
            dimension_semantics=("parallel",)),
    )(*args)


def _node_body(h_ref, macc_ref, pacc_ref, pos_ref,
               u1h_ref, u1m_ref, ub1_ref, ug1_ref, ubn1_ref,
               u2_ref, ub2_ref, ug2_ref, ubn2_ref,
               wd_ref, ws_ref, h_out, pos_out, *outs, last):
    h = h_ref[...]
    macc = macc_ref[...][0] + macc_ref[...][1]
    pacc = pacc_ref[...]
    cnt = jnp.maximum(pacc[:, 3:4], 1.0)
    pos_aggr = pacc[:, :3] / cnt
    x = (jnp.dot(h, u1h_ref[...], preferred_element_type=jnp.float32)
         + jnp.dot(macc, u1m_ref[...], preferred_element_type=jnp.float32)
         + ub1_ref[...])
    x = jax.nn.relu(_ln2d(x, ug1_ref[...], ubn1_ref[...]))
    x = jnp.dot(x, u2_ref[...], preferred_element_type=jnp.float32) + ub2_ref[...]
    x = jax.nn.relu(_ln2d(x, ug2_ref[...], ubn2_ref[...]))
    hn = h + x
    h_out[...] = hn
    pos = pos_ref[...]
    zero = jnp.zeros((pos.shape[0], 1), jnp.float32)
    posn = pos + jnp.concatenate([pos_aggr, zero], axis=-1)
    pos_out[...] = posn
    if not last:
        ta_out, tb_out = outs
        ta_out[...] = jnp.dot(hn, wd_ref[...], preferred_element_type=jnp.float32)
        tb_out[...] = jnp.dot(hn, ws_ref[...], preferred_element_type=jnp.float32)


def _node_call(h, macc2, pacc, pos4, lp, nxt_wd, nxt_ws, last):
    grid = N // NBLK
    row = lambda i: (i, 0)
    cst = lambda i: (0, 0)
    args = [
        h, macc2, pacc, pos4,
        lp["upd_l1"]["w"][:D, :], lp["upd_l1"]["w"][D:, :], lp["upd_l1"]["b"][None, :],
        lp["upd_n1"]["g"][None, :], lp["upd_n1"]["b"][None, :],
        lp["upd_l2"]["w"], lp["upd_l2"]["b"][None, :],
        lp["upd_n2"]["g"][None, :], lp["upd_n2"]["b"][None, :],
        nxt_wd, nxt_ws,
    ]
    in_specs = [
        pl.BlockSpec((NBLK, D), row),
        pl.BlockSpec((NC, NBLK, D), lambda i: (0, i, 0)),
        pl.BlockSpec((NBLK, 4), row),
        pl.BlockSpec((NBLK, 4), row),
        pl.BlockSpec((D, D), cst), pl.BlockSpec((D, D), cst), pl.BlockSpec((1, D), cst),
        pl.BlockSpec((1, D), cst), pl.BlockSpec((1, D), cst),
        pl.BlockSpec((D, D), cst), pl.BlockSpec((1, D), cst),
        pl.BlockSpec((1, D), cst), pl.BlockSpec((1, D), cst),
        pl.BlockSpec((D, D), cst), pl.BlockSpec((D, D), cst),
    ]
    out_specs = [pl.BlockSpec((NBLK, D), row), pl.BlockSpec((NBLK, 4), row)]
    out_shape = [jax.ShapeDtypeStruct((N, D), jnp.float32),
                 jax.ShapeDtypeStruct((N, 4), jnp.float32)]
    if not last:
        out_specs += [pl.BlockSpec((NBLK, D), row), pl.BlockSpec((NBLK, D), row)]
        out_shape += [jax.ShapeDtypeStruct((N, D), jnp.float32),
                      jax.ShapeDtypeStruct((N, D), jnp.float32)]
    res = pl.pallas_call(
        functools.partial(_node_body, last=last),
        grid=(grid,),
        in_specs=in_specs,
        out_specs=out_specs,
        out_shape=out_shape,
        compiler_params=pltpu.CompilerParams(
            dimension_semantics=("parallel",)),
    )(*args)
    if last:
        return res[0], res[1], None, None
    return res


def _pool_body(h_ref, bt_ref, p1_ref, pb1_ref, p2_ref, pb2_ref,
               out_ref, acc_ref):
    i = pl.program_id(0)

    @pl.when(i == 0)
    def _():
        acc_ref[...] = jnp.zeros_like(acc_ref)

    bt = bt_ref[...].reshape(1, -1)
    gi = jax.lax.broadcasted_iota(jnp.int32, (G, 1), 0)
    mask = (bt == gi).astype(jnp.float32)
    acc_ref[...] += jnp.dot(mask, h_ref[...], preferred_element_type=jnp.float32)

    @pl.when(i == pl.num_programs(0) - 1)
    def _():
        pooled = acc_ref[...]
        x = jax.nn.relu(jnp.dot(pooled, p1_ref[...],
                                preferred_element_type=jnp.float32) + pb1_ref[...])
        out_ref[...] = jnp.dot(x, p2_ref[...],
                               preferred_element_type=jnp.float32) + pb2_ref[...]


def _pool_call(h, batch3d, params):
    grid = N // NBLK
    cst = lambda i: (0, 0)
    return pl.pallas_call(
        _pool_body,
        grid=(grid,),
        in_specs=[
            pl.BlockSpec((NBLK, D), lambda i: (i, 0)),
            pl.BlockSpec((1, 1, NBLK), lambda i: (i, 0, 0)),
            pl.BlockSpec((D, D), cst), pl.BlockSpec((1, D), cst),
            pl.BlockSpec((D, 1), cst), pl.BlockSpec((1, 1), cst),
        ],
        out_specs=pl.BlockSpec((G, 1), cst),
        out_shape=jax.ShapeDtypeStruct((G, 1), jnp.float32),
        scratch_shapes=[pltpu.VMEM((G, D), jnp.float32)],
        compiler_params=pltpu.CompilerParams(
            dimension_semantics=("arbitrary",)),
    )(h, batch3d, params["pred_l1"]["w"], params["pred_l1"]["b"][None, :],
      params["pred_l2"]["w"], params["pred_l2"]["b"][None, :])


def kernel(atoms, pos, edge_index, batch, params):
    src = edge_index[0]
    dst = edge_index[1]
    atoms2d = atoms[:, None]
    batch3d = batch.reshape(N // NBLK, 1, NBLK)
    pos4 = jnp.pad(pos, ((0, 0), (0, 1)))
    zer = jnp.zeros((NPTL, D), jnp.float32)

    layers = params["layers"]
    h, ta, tb = _init_call(atoms2d, params["emb"],
                           layers[0]["msg_l1"]["w"][:D, :],
                           layers[0]["msg_l1"]["w"][D:2 * D, :])

    for li in range(L):
        lp = layers[li]
        xpre, auxflat = _sc_gather(ta, tb, pos4.reshape(-1), dst, src)
        msg, pd = _edge_call(xpre, auxflat.reshape(E, 4), lp)
        maccf = _sc_mscat(msg, dst, zer)
        pdpart = _sc_pscat(pd, dst)
        macc2 = maccf.reshape(NC, N, D)
        pr = _pdred_call(pdpart.reshape(NW, N4P))
        pacc = pr.reshape(-1)[:N * 4].reshape(N, 4)

        last = li == L - 1
        if last:
            nxt_wd = jnp.zeros((D, D), jnp.float32)
            nxt_ws = jnp.zeros((D, D), jnp.float32)
        else:
            nxt_wd = layers[li + 1]["msg_l1"]["w"][:D, :]
            nxt_ws = layers[li + 1]["msg_l1"]["w"][D:2 * D, :]
        h, pos4, ta, tb = _node_call(h, macc2, pacc, pos4, lp, nxt_wd, nxt_ws, last)

    return _pool_call(h, batch3d, params)


# speedup vs baseline: 1.0689x; 1.0002x over previous
import functools
import jax
import jax.numpy as jnp
from jax import lax
from jax.experimental import pallas as pl
from jax.experimental.pallas import tpu as pltpu
from jax.experimental.pallas import tpu_sc as plsc

D = 128
N = 10000
E = 320000
G = 64
VOCAB = 10
L = 5

EBLK = 2560
NBLK = 2000
EPS = 1e-5

NC = 2
NS = 16
NW = NC * NS
EPW = E // NW
GC = 80
GCH = EPW // GC
SC2 = 80
SCH = EPW // SC2
NPT0 = 624
NPTL = N - 15 * NPT0
N4P = 40960
PDB = 8192
PC = 400


def _ln2d(x, g, b):
    m = jnp.mean(x, axis=-1, keepdims=True)
    v = jnp.mean((x - m) ** 2, axis=-1, keepdims=True)
    return (x - m) * jax.lax.rsqrt(v + EPS) * g + b


def _gather_chunk_compute(bufa, bufb, auxb, dstv, srcv, posv, t, lanes):
    def row(i, cy):
        for k in range(8):
            sl = pl.ds(k * 16, 16)
            bufa[i, sl] = bufa[i, sl] + bufb[i, sl]
        return cy

    lax.fori_loop(0, GC, row, 0)

    def grp(j, cy):
        d16 = dstv[pl.ds(t * GC + j * 16, 16)] * 4
        s16 = srcv[pl.ds(t * GC + j * 16, 16)] * 4
        e16 = (lanes + j * 16) * 4
        d2 = jnp.zeros((16,), jnp.float32)
        for comp in range(3):
            pdv = plsc.load_gather(posv, [d16 + comp])
            psv = plsc.load_gather(posv, [s16 + comp])
            df = pdv - psv
            plsc.store_scatter(auxb, [e16 + comp], df)
            d2 = d2 + df * df
        plsc.store_scatter(auxb, [e16 + 3], d2)
        return cy

    lax.fori_loop(0, GC // 16, grp, 0)


def _gather_body(ta, tb, posf, dsth, srch, xout, auxout,
                 dstv, srcv, posv, bufa0, bufb0, bufa1, bufb1, auxb0, auxb1,
                 ga0, gb0, ga1, gb1, wx0, wa0, wx1, wa1):
    c = lax.axis_index("c")
    s = lax.axis_index("s")
    wid = c * NS + s
    ebase = wid * EPW
    pltpu.sync_copy(posf, posv)
    pltpu.sync_copy(dsth.at[pl.ds(ebase, EPW)], dstv)
    pltpu.sync_copy(srch.at[pl.ds(ebase, EPW)], srcv)
    lanes = lax.iota(jnp.int32, 16)

    def g_copies(t, ba, bb, sa, sb):
        off = pl.ds(t * GC, GC)
        return (pltpu.make_async_copy(ta.at[dstv.at[off]], ba, sa),
                pltpu.make_async_copy(tb.at[srcv.at[off]], bb, sb))

    def w_copies(t, ba, ab, sx, sxa):
        base = ebase + t * GC
        return (pltpu.make_async_copy(ba, xout.at[pl.ds(base, GC)], sx),
                pltpu.make_async_copy(ab, auxout.at[pl.ds(base * 4, GC * 4)], sxa))

    a0, b0 = g_copies(0, bufa0, bufb0, ga0, gb0)
    a0.start()
    b0.start()

    def pair(i, carry):
        t0 = 2 * i
        t1 = 2 * i + 1
        t2 = jnp.minimum(t0 + 2, GCH - 1)
        a0, b0 = g_copies(t0, bufa0, bufb0, ga0, gb0)
        a0.wait()
        b0.wait()

        @pl.when(i > 0)
        def _():
            x1, xa1 = w_copies(t0 - 1, bufa1, auxb1, wx1, wa1)
            x1.wait()
            xa1.wait()

        a1, b1 = g_copies(t1, bufa1, bufb1, ga1, gb1)
        a1.start()
        b1.start()
        _gather_chunk_compute(bufa0, bufb0, auxb0, dstv, srcv, posv, t0, lanes)
        x0, xa0 = w_copies(t0, bufa0, auxb0, wx0, wa0)
        x0.start()
        xa0.start()
        a1, b1 = g_copies(t1, bufa1, bufb1, ga1, gb1)
        a1.wait()
        b1.wait()
        x0, xa0 = w_copies(t0, bufa0, auxb0, wx0, wa0)
        x0.wait()
        xa0.wait()
        a2, b2 = g_copies(t2, bufa0, bufb0, ga0, gb0)
        a2.start()
        b2.start()
        _gather_chunk_compute(bufa1, bufb1, auxb1, dstv, srcv, posv, t1, lanes)
        x1, xa1 = w_copies(t1, bufa1, auxb1, wx1, wa1)
        x1.start()
        xa1.start()
        return carry

    lax.fori_loop(0, GCH // 2, pair, 0)
    af, bf = g_copies(GCH - 1, bufa0, bufb0, ga0, gb0)
    af.wait()
    bf.wait()
    xf, xaf = w_copies(GCH - 2, bufa1, auxb1, wx1, wa1)
    xf.wait()
    xaf.wait()
    _gather_chunk_compute(bufa0, bufb0, auxb0, dstv, srcv, posv, GCH - 1, lanes)
    base = ebase + (GCH - 1) * GC
    pltpu.sync_copy(bufa0, xout.at[pl.ds(base, GC)])
    pltpu.sync_copy(auxb0, auxout.at[pl.ds(base * 4, GC * 4)])


def _sc_gather(ta, tb, posflat, dst, src):
    mesh = plsc.VectorSubcoreMesh(core_axis_name="c", subcore_axis_name="s")
    f = pl.kernel(
        _gather_body,
        mesh=mesh,
        out_type=[
            jax.ShapeDtypeStruct((E, D), jnp.float32),
            jax.ShapeDtypeStruct((E * 4,), jnp.float32),
        ],
        scratch_types=[
            pltpu.VMEM((EPW,), jnp.int32),
            pltpu.VMEM((EPW,), jnp.int32),
            pltpu.VMEM((N * 4,), jnp.float32),
            pltpu.VMEM((GC, D), jnp.float32),
            pltpu.VMEM((GC, D), jnp.float32),
            pltpu.VMEM((GC, D), jnp.float32),
            pltpu.VMEM((GC, D), jnp.float32),
            pltpu.VMEM((GC * 4,), jnp.float32),
            pltpu.VMEM((GC * 4,), jnp.float32),
            pltpu.SemaphoreType.DMA,
            pltpu.SemaphoreType.DMA,
            pltpu.SemaphoreType.DMA,
            pltpu.SemaphoreType.DMA,
            pltpu.SemaphoreType.DMA,
            pltpu.SemaphoreType.DMA,
            pltpu.SemaphoreType.DMA,
            pltpu.SemaphoreType.DMA,
        ],
        compiler_params=pltpu.CompilerParams(needs_layout_passes=False),
    )
    return f(ta, tb, posflat, dst, src)


def _mscat_body(msgh, dsth, zerh, mout, dsti0, dsti1, msgbuf0, msgbuf1, acc,
                sm0, sm1):
    c = lax.axis_index("c")
    s = lax.axis_index("s")
    wid = c * NS + s

    @pl.when(s < 15)
    def _():
        pltpu.sync_copy(zerh.at[pl.ds(0, NPT0)], acc.at[pl.ds(s * NPT0, NPT0)])

    @pl.when(s == 15)
    def _():
        pltpu.sync_copy(zerh, acc.at[pl.ds(15 * NPT0, NPTL)])

    plsc.subcore_barrier()

    pltpu.sync_copy(dsth.at[pl.ds(wid * EPW, SC2)], dsti0)
    pltpu.make_async_copy(msgh.at[pl.ds(wid * EPW, SC2)], msgbuf0, sm0).start()

    def pair(i, carry):
        t0 = 2 * i
        t1 = 2 * i + 1
        t2 = jnp.minimum(t0 + 2, SCH - 1)
        b0 = wid * EPW + t0 * SC2
        b1 = wid * EPW + t1 * SC2
        b2 = wid * EPW + t2 * SC2
        pltpu.make_async_copy(msgh.at[pl.ds(b0, SC2)], msgbuf0, sm0).wait()
        pltpu.sync_copy(dsth.at[pl.ds(b1, SC2)], dsti1)
        pltpu.make_async_copy(msgh.at[pl.ds(b1, SC2)], msgbuf1, sm1).start()
        pltpu.sync_copy(msgbuf0, acc.at[dsti0], add=True)
        pltpu.make_async_copy(msgh.at[pl.ds(b1, SC2)], msgbuf1, sm1).wait()
        pltpu.sync_copy(dsth.at[pl.ds(b2, SC2)], dsti0)
        pltpu.make_async_copy(msgh.at[pl.ds(b2, SC2)], msgbuf0, sm0).start()
        pltpu.sync_copy(msgbuf1, acc.at[dsti1], add=True)
        return carry

    lax.fori_loop(0, SCH // 2, pair, 0)
    pltpu.make_async_copy(msgh.at[pl.ds(wid * EPW + (SCH - 1) * SC2, SC2)],
                          msgbuf0, sm0).wait()
    pltpu.sync_copy(msgbuf0, acc.at[dsti0], add=True)
    plsc.subcore_barrier()

    @pl.when(s < 15)
    def _():
        pltpu.sync_copy(acc.at[pl.ds(s * NPT0, NPT0)],
                        mout.at[pl.ds(c * N + s * NPT0, NPT0)])

    @pl.when(s == 15)
    def _():
        pltpu.sync_copy(acc.at[pl.ds(15 * NPT0, NPTL)],
                        mout.at[pl.ds(c * N + 15 * NPT0, NPTL)])


def _sc_mscat(msg, dst, zer):
    mesh = plsc.VectorSubcoreMesh(core_axis_name="c", subcore_axis_name="s")
    f = pl.kernel(
        _mscat_body,
        mesh=mesh,
        out_type=jax.ShapeDtypeStruct((NC * N, D), jnp.float32),
        scratch_types=[
            pltpu.VMEM((SC2,), jnp.int32),
            pltpu.VMEM((SC2,), jnp.int32),
            pltpu.VMEM((SC2, D), jnp.float32),
            pltpu.VMEM((SC2, D), jnp.float32),
            pltpu.VMEM_SHARED((N, D), jnp.float32),
            pltpu.SemaphoreType.DMA,
            pltpu.SemaphoreType.DMA,
        ],
        compiler_params=pltpu.CompilerParams(needs_layout_passes=False),
    )
    return f(msg, dst, zer)


def _pscat_body(pdh, dsth, pout, dstv, pdbuf, pdacc):
    c = lax.axis_index("c")
    s = lax.axis_index("s")
    wid = c * NS + s
    lanes = lax.iota(jnp.int32, 16)

    def z1(i, cy):
        pdacc[pl.ds(i * 16, 16)] = jnp.zeros((16,), jnp.float32)
        return cy

    lax.fori_loop(0, N4P // 16, z1, 0)
    pltpu.sync_copy(dsth.at[pl.ds(wid * EPW, EPW)], dstv)

    def chunk(t, carry):
        base = wid * EPW + t * PC
        pltpu.sync_copy(pdh.at[pl.ds(base, PC)], pdbuf)

        def grp(j, cy):
            d16 = dstv[pl.ds(t * PC + j * 16, 16)] * 4
            e16 = lanes + j * 16
            for comp in range(4):
                v = plsc.load_gather(pdbuf, [e16, jnp.full((16,), comp, jnp.int32)])
                plsc.addupdate_scatter(pdacc, [d16 + comp], v)
            return cy

        lax.fori_loop(0, PC // 16, grp, 0)
        return carry

    lax.fori_loop(0, EPW // PC, chunk, 0)
    pltpu.sync_copy(pdacc, pout.at[pl.ds(wid * N4P, N4P)])


def _sc_pscat(pd8, dst):
    mesh = plsc.VectorSubcoreMesh(core_axis_name="c", subcore_axis_name="s")
    f = pl.kernel(
        _pscat_body,
        mesh=mesh,
        out_type=jax.ShapeDtypeStruct((NW * N4P,), jnp.float32),
        scratch_types=[
            pltpu.VMEM((EPW,), jnp.int32),
            pltpu.VMEM((PC, 8), jnp.float32),
            pltpu.VMEM((N4P,), jnp.float32),
        ],
        compiler_params=pltpu.CompilerParams(needs_layout_passes=False),
    )
    return f(pd8, dst)


def _pdred_body(p_ref, o_ref):
    o_ref[...] = jnp.sum(p_ref[...], axis=0, keepdims=True)[None]


def _pdred_call(p2d):
    grid = N4P // PDB
    return pl.pallas_call(
        _pdred_body,
        grid=(grid,),
        in_specs=[pl.BlockSpec((NW, PDB), lambda i: (0, i))],
        out_specs=pl.BlockSpec((1, 1, PDB), lambda i: (i, 0, 0)),
        out_shape=jax.ShapeDtypeStruct((grid, 1, PDB), jnp.float32),
        compiler_params=pltpu.CompilerParams(
            dimension_semantics=("parallel",)),
    )(p2d)


def _init_body(atoms_ref, emb_ref, wd_ref, ws_ref, h_ref, ta_ref, tb_ref):
    at = atoms_ref[...]
    oh = (at == jax.lax.broadcasted_iota(jnp.int32, (1, VOCAB), 1)).astype(jnp.float32)
    h = jnp.dot(oh, emb_ref[...], preferred_element_type=jnp.float32)
    h_ref[...] = h
    ta_ref[...] = jnp.dot(h, wd_ref[...], preferred_element_type=jnp.float32)
    tb_ref[...] = jnp.dot(h, ws_ref[...], preferred_element_type=jnp.float32)


def _init_call(atoms2d, emb, wd, ws):
    grid = N // NBLK
    cst = lambda i: (0, 0)
    return pl.pallas_call(
        _init_body,
        grid=(grid,),
        in_specs=[
            pl.BlockSpec((NBLK, 1), lambda i: (i, 0)),
            pl.BlockSpec((VOCAB, D), cst),
            pl.BlockSpec((D, D), cst),
            pl.BlockSpec((D, D), cst),
        ],
        out_specs=[
            pl.BlockSpec((NBLK, D), lambda i: (i, 0)),
            pl.BlockSpec((NBLK, D), lambda i: (i, 0)),
            pl.BlockSpec((NBLK, D), lambda i: (i, 0)),
        ],
        out_shape=[
            jax.ShapeDtypeStruct((N, D), jnp.float32),
            jax.ShapeDtypeStruct((N, D), jnp.float32),
            jax.ShapeDtypeStruct((N, D), jnp.float32),
        ],
        compiler_params=pltpu.CompilerParams(
            dimension_semantics=("parallel",)),
    )(atoms2d, emb, wd, ws)


def _edge_body(xp_ref, aux_ref, wdist_ref, b1_ref, g1_ref, bn1_ref,
               w2_ref, b2_ref, g2_ref, bn2_ref,
               p1_ref, pb1_ref, pg_ref, pbn_ref, p2_ref, pb2_ref,
               msg_ref, pd_ref):
    xp = xp_ref[...]
    aux = aux_ref[...]
    diff = aux[:, :3]
    dist = jnp.sqrt(aux[:, 3:4])
    x = xp + dist * wdist_ref[...] + b1_ref[...]
    x = jax.nn.relu(_ln2d(x, g1_ref[...], bn1_ref[...]))
    x = jnp.dot(x, w2_ref[...], preferred_element_type=jnp.float32) + b2_ref[...]
    msg = jax.nn.relu(_ln2d(x, g2_ref[...], bn2_ref[...]))
    msg_ref[...] = msg
    p = jnp.dot(msg, p1_ref[...], preferred_element_type=jnp.float32) + pb1_ref[...]
    p = jax.nn.relu(_ln2d(p, pg_ref[...], pbn_ref[...]))
    pw = jnp.dot(p, p2_ref[...], preferred_element_type=jnp.float32) + pb2_ref[...]
    ones = jnp.ones((xp.shape[0], 1), jnp.float32)
    z4 = jnp.zeros((xp.shape[0], 4), jnp.float32)
    pd_ref[...] = jnp.concatenate([diff * pw, ones, z4], axis=-1)


def _edge_call(xpre, aux, lp):
    grid = E // EBLK
    row = lambda i: (i, 0)
    cst = lambda i: (0, 0)
    wdist = lp["msg_l1"]["w"][2 * D:2 * D + 1, :]
    args = [
        xpre, aux,
        wdist, lp["msg_l1"]["b"][None, :], lp["msg_n1"]["g"][None, :], lp["msg_n1"]["b"][None, :],
        lp["msg_l2"]["w"], lp["msg_l2"]["b"][None, :], lp["msg_n2"]["g"][None, :], lp["msg_n2"]["b"][None, :],
        lp["pos_l1"]["w"], lp["pos_l1"]["b"][None, :], lp["pos_n1"]["g"][None, :], lp["pos_n1"]["b"][None, :],
        lp["pos_l2"]["w"], lp["pos_l2"]["b"][None, :],
    ]
    in_specs = [
        pl.BlockSpec((EBLK, D), row),
        pl.BlockSpec((EBLK, 4), row),
        pl.BlockSpec((1, D), cst), pl.BlockSpec((1, D), cst),
        pl.BlockSpec((1, D), cst), pl.BlockSpec((1, D), cst),
        pl.BlockSpec((D, D), cst), pl.BlockSpec((1, D), cst),
        pl.BlockSpec((1, D), cst), pl.BlockSpec((1, D), cst),
        pl.BlockSpec((D, D), cst), pl.BlockSpec((1, D), cst),
        pl.BlockSpec((1, D), cst), pl.BlockSpec((1, D), cst),
        pl.BlockSpec((D, 1), cst), pl.BlockSpec((1, 1), cst),
    ]
    return pl.pallas_call(
        _edge_body,
        grid=(grid,),
        in_specs=in_specs,
        out_specs=[
            pl.BlockSpec((EBLK, D), row),
            pl.BlockSpec((EBLK, 8), row),
        ],
        out_shape=[
            jax.ShapeDtypeStruct((E, D), jnp.float32),
            jax.ShapeDtypeStruct((E, 8), jnp.float32),
        ],
        compiler_params=pltpu.CompilerParams(
            dimension_semantics=("parallel",)),
    )(*args)


def _node_body(h_ref, macc_ref, pacc_ref, pos_ref,
               u1h_ref, u1m_ref, ub1_ref, ug1_ref, ubn1_ref,
               u2_ref, ub2_ref, ug2_ref, ubn2_ref,
               wd_ref, ws_ref, h_out, pos_out, *outs, last):
    h = h_ref[...]
    macc = macc_ref[...][0] + macc_ref[...][1]
    pacc = pacc_ref[...]
    cnt = jnp.maximum(pacc[:, 3:4], 1.0)
    pos_aggr = pacc[:, :3] / cnt
    x = (jnp.dot(h, u1h_ref[...], preferred_element_type=jnp.float32)
         + jnp.dot(macc, u1m_ref[...], preferred_element_type=jnp.float32)
         + ub1_ref[...])
    x = jax.nn.relu(_ln2d(x, ug1_ref[...], ubn1_ref[...]))
    x = jnp.dot(x, u2_ref[...], preferred_element_type=jnp.float32) + ub2_ref[...]
    x = jax.nn.relu(_ln2d(x, ug2_ref[...], ubn2_ref[...]))
    hn = h + x
    h_out[...] = hn
    pos = pos_ref[...]
    zero = jnp.zeros((pos.shape[0], 1), jnp.float32)
    posn = pos + jnp.concatenate([pos_aggr, zero], axis=-1)
    pos_out[...] = posn
    if not last:
        ta_out, tb_out = outs
        ta_out[...] = jnp.dot(hn, wd_ref[...], preferred_element_type=jnp.float32)
        tb_out[...] = jnp.dot(hn, ws_ref[...], preferred_element_type=jnp.float32)


def _node_call(h, macc2, pacc, pos4, lp, nxt_wd, nxt_ws, last):
    grid = N // NBLK
    row = lambda i: (i, 0)
    cst = lambda i: (0, 0)
    args = [
        h, macc2, pacc, pos4,
        lp["upd_l1"]["w"][:D, :], lp["upd_l1"]["w"][D:, :], lp["upd_l1"]["b"][None, :],
        lp["upd_n1"]["g"][None, :], lp["upd_n1"]["b"][None, :],
        lp["upd_l2"]["w"], lp["upd_l2"]["b"][None, :],
        lp["upd_n2"]["g"][None, :], lp["upd_n2"]["b"][None, :],
        nxt_wd, nxt_ws,
    ]
    in_specs = [
        pl.BlockSpec((NBLK, D), row),
        pl.BlockSpec((NC, NBLK, D), lambda i: (0, i, 0)),
        pl.BlockSpec((NBLK, 4), row),
        pl.BlockSpec((NBLK, 4), row),
        pl.BlockSpec((D, D), cst), pl.BlockSpec((D, D), cst), pl.BlockSpec((1, D), cst),
        pl.BlockSpec((1, D), cst), pl.BlockSpec((1, D), cst),
        pl.BlockSpec((D, D), cst), pl.BlockSpec((1, D), cst),
        pl.BlockSpec((1, D), cst), pl.BlockSpec((1, D), cst),
        pl.BlockSpec((D, D), cst), pl.BlockSpec((D, D), cst),
    ]
    out_specs = [pl.BlockSpec((NBLK, D), row), pl.BlockSpec((NBLK, 4), row)]
    out_shape = [jax.ShapeDtypeStruct((N, D), jnp.float32),
                 jax.ShapeDtypeStruct((N, 4), jnp.float32)]
    if not last:
        out_specs += [pl.BlockSpec((NBLK, D), row), pl.BlockSpec((NBLK, D), row)]
        out_shape += [jax.ShapeDtypeStruct((N, D), jnp.float32),
                      jax.ShapeDtypeStruct((N, D), jnp.float32)]
    res = pl.pallas_call(
        functools.partial(_node_body, last=last),
        grid=(grid,),
        in_specs=in_specs,
        out_specs=out_specs,
        out_shape=out_shape,
        compiler_params=pltpu.CompilerParams(
            dimension_semantics=("parallel",)),
    )(*args)
    if last:
        return res[0], res[1], None, None
    return res


def _pool_body(h_ref, bt_ref, p1_ref, pb1_ref, p2_ref, pb2_ref,
               out_ref, acc_ref):
    i = pl.program_id(0)

    @pl.when(i == 0)
    def _():
        acc_ref[...] = jnp.zeros_like(acc_ref)

    bt = bt_ref[...].reshape(1, -1)
    gi = jax.lax.broadcasted_iota(jnp.int32, (G, 1), 0)
    mask = (bt == gi).astype(jnp.float32)
    acc_ref[...] += jnp.dot(mask, h_ref[...], preferred_element_type=jnp.float32)

    @pl.when(i == pl.num_programs(0) - 1)
    def _():
        pooled = acc_ref[...]
        x = jax.nn.relu(jnp.dot(pooled, p1_ref[...],
                                preferred_element_type=jnp.float32) + pb1_ref[...])
        out_ref[...] = jnp.dot(x, p2_ref[...],
                               preferred_element_type=jnp.float32) + pb2_ref[...]


def _pool_call(h, batch3d, params):
    grid = N // NBLK
    cst = lambda i: (0, 0)
    return pl.pallas_call(
        _pool_body,
        grid=(grid,),
        in_specs=[
            pl.BlockSpec((NBLK, D), lambda i: (i, 0)),
            pl.BlockSpec((1, 1, NBLK), lambda i: (i, 0, 0)),
            pl.BlockSpec((D, D), cst), pl.BlockSpec((1, D), cst),
            pl.BlockSpec((D, 1), cst), pl.BlockSpec((1, 1), cst),
        ],
        out_specs=pl.BlockSpec((G, 1), cst),
        out_shape=jax.ShapeDtypeStruct((G, 1), jnp.float32),
        scratch_shapes=[pltpu.VMEM((G, D), jnp.float32)],
        compiler_params=pltpu.CompilerParams(
            dimension_semantics=("arbitrary",)),
    )(h, batch3d, params["pred_l1"]["w"], params["pred_l1"]["b"][None, :],
      params["pred_l2"]["w"], params["pred_l2"]["b"][None, :])


def kernel(atoms, pos, edge_index, batch, params):
    src = edge_index[0]
    dst = edge_index[1]
    atoms2d = atoms[:, None]
    batch3d = batch.reshape(N // NBLK, 1, NBLK)
    pos4 = jnp.pad(pos, ((0, 0), (0, 1)))
    zer = jnp.zeros((NPTL, D), jnp.float32)

    layers = params["layers"]
    h, ta, tb = _init_call(atoms2d, params["emb"],
                           layers[0]["msg_l1"]["w"][:D, :],
                           layers[0]["msg_l1"]["w"][D:2 * D, :])

    for li in range(L):
        lp = layers[li]
        xpre, auxflat = _sc_gather(ta, tb, pos4.reshape(-1), dst, src)
        msg, pd = _edge_call(xpre, auxflat.reshape(E, 4), lp)
        maccf = _sc_mscat(msg, dst, zer)
        pdpart = _sc_pscat(pd, dst)
        macc2 = maccf.reshape(NC, N, D)
        pr = _pdred_call(pdpart.reshape(NW, N4P))
        pacc = pr.reshape(-1)[:N * 4].reshape(N, 4)

        last = li == L - 1
        if last:
            nxt_wd = jnp.zeros((D, D), jnp.float32)
            nxt_ws = jnp.zeros((D, D), jnp.float32)
        else:
            nxt_wd = layers[li + 1]["msg_l1"]["w"][:D, :]
            nxt_ws = layers[li + 1]["msg_l1"]["w"][D:2 * D, :]
        h, pos4, ta, tb = _node_call(h, macc2, pacc, pos4, lp, nxt_wd, nxt_ws, last)

    return _pool_call(h, batch3d, params)
